# baseline (speedup 1.0000x reference)
import jax
import jax.numpy as jnp
from jax import lax
from jax.experimental import pallas as pl
from jax.experimental.pallas import tpu as pltpu


def _net_body(x_ref, a_ref, e_ref, et_ref, wsi_ref, wsj_ref, bs_ref,
              wsc_ref, wnx_ref, wni_ref, wnj_ref, bn_ref, wd_ref, bd_ref,
              out_ref, s_scr, vt_scr, u_scr, di_scr, dj_scr, pi_scr, pjt_scr):
    N = a_ref.shape[1]
    C = bs_ref.shape[1]
    f32 = jnp.float32

    xb = x_ref[0]

    u_scr[...] = jnp.dot(xb, wsi_ref[...],
                         preferred_element_type=f32) + bs_ref[...]
    v = jnp.dot(xb, wsj_ref[...], preferred_element_type=f32)
    vt_scr[...] = v.T

    ones_col = jnp.ones((N, 1), f32)
    ones_row = jnp.ones((1, N), f32)
    iota_cc = lax.broadcasted_iota(jnp.int32, (C, 16), 0)
    iota_8c = lax.broadcasted_iota(jnp.int32, (16, C), 1)
    eight_col = lax.broadcasted_iota(jnp.int32, (C, 16), 1)
    eight_row = lax.broadcasted_iota(jnp.int32, (16, C), 0)

    TI = 40

    def pass1_tile(t, _):
        i0 = t * TI
        for j0 in range(0, N, 128):
            W = min(128, N - j0)
            eb = e_ref[0, pl.ds(i0, TI), j0:j0 + W]
            etb = et_ref[0, pl.ds(i0, TI), j0:j0 + W]
            ab = a_ref[0, pl.ds(i0, TI), j0:j0 + W]
            acc_i = None
            for c in range(C):
                ub = jnp.broadcast_to(u_scr[pl.ds(i0, TI), c:c + 1],
                                      (TI, W))
                vb = jnp.broadcast_to(vt_scr[c:c + 1, j0:j0 + W], (TI, W))
                s_c = jnp.maximum(ub + vb + wsc_ref[0, c] * eb
                                  + wsc_ref[1, c] * etb, 0.0) * ab
                s_scr[c, pl.ds(i0, TI), j0:j0 + W] = s_c
                if acc_i is None:
                    acc_i = wsc_ref[2, c] * s_c
                    acc_j = wsc_ref[3, c] * s_c
                else:
                    acc_i = acc_i + wsc_ref[2, c] * s_c
                    acc_j = acc_j + wsc_ref[3, c] * s_c
            di_scr[pl.ds(i0, TI), j0:j0 + W] = acc_i
            dj_scr[pl.ds(i0, TI), j0:j0 + W] = acc_j
        return 0

    lax.fori_loop(0, N // TI, pass1_tile, 0)

    bai = wsc_ref[4, 0]
    baj = wsc_ref[4, 1]
    di_scr[...] = jax.nn.sigmoid(di_scr[...] + bai)
    dj_scr[...] = jax.nn.sigmoid(dj_scr[...] + baj)

    pi_scr[...] = jnp.zeros((N, C), f32)
    pjt_scr[...] = jnp.zeros((C, N), f32)

    def pass2(c0):
        sigi = di_scr[...]
        sigj = dj_scr[...]
        pcl = []
        prl = []
        for k in range(16):
            s_k = s_scr[c0 + k]
            pcl.append(jnp.dot(s_k * sigi, ones_col,
                               preferred_element_type=f32))
            prl.append(jnp.dot(ones_row, s_k * sigj,
                               preferred_element_type=f32))
        pcols = jnp.concatenate(pcl, axis=1)
        prows = jnp.concatenate(prl, axis=0)
        ohrows = (iota_8c == c0 + eight_row).astype(f32)
        pi_scr[...] = pi_scr[...] + jnp.dot(pcols, ohrows,
                                            preferred_element_type=f32)
        pjt_scr[...] = pjt_scr[...] + lax.dot_general(
            ohrows, prows, (((0,), (0,)), ((), ())),
            preferred_element_type=f32)

    pass2(0)
    pass2(16)

    xo = (jnp.dot(xb, wnx_ref[...], preferred_element_type=f32)
          + jnp.dot(pi_scr[...], wni_ref[...], preferred_element_type=f32)
          + lax.dot_general(pjt_scr[...], wnj_ref[...],
                            (((0,), (0,)), ((), ())),
                            preferred_element_type=f32)
          + bn_ref[...])
    out_ref[0] = jnp.dot(xo, wd_ref[...], preferred_element_type=f32) \
        + bd_ref[...]


def kernel(x, a, e, Ws, bs, Wai, bai, Waj, baj, Wn, bn, We, be, Wd, bd):
    B, N, F = x.shape
    C = Ws.shape[1]
    LBL = Wd.shape[1]
    f32 = jnp.float32

    e2 = e[..., 0]
    et2 = jnp.swapaxes(e2, 1, 2)
    wsi = Ws[:F]
    wsj = Ws[F:2 * F]
    brow = jnp.zeros((C,), f32).at[0].set(bai[0]).at[1].set(baj[0])
    wsc = jnp.stack([Ws[2 * F], Ws[2 * F + 1], Wai[:, 0], Waj[:, 0], brow],
                    axis=0)
    wnx = Wn[:F]
    wni = Wn[F:F + C]
    wnj = Wn[F + C:]

    out = pl.pallas_call(
        _net_body,
        grid=(B,),
        in_specs=[
            pl.BlockSpec((1, N, F), lambda b: (b, 0, 0)),
            pl.BlockSpec((1, N, N), lambda b: (b, 0, 0)),
            pl.BlockSpec((1, N, N), lambda b: (b, 0, 0)),
            pl.BlockSpec((1, N, N), lambda b: (b, 0, 0)),
            pl.BlockSpec((F, C), lambda b: (0, 0)),
            pl.BlockSpec((F, C), lambda b: (0, 0)),
            pl.BlockSpec((1, C), lambda b: (0, 0)),
            pl.BlockSpec(memory_space=pltpu.SMEM),
            pl.BlockSpec((F, F), lambda b: (0, 0)),
            pl.BlockSpec((C, F), lambda b: (0, 0)),
            pl.BlockSpec((C, F), lambda b: (0, 0)),
            pl.BlockSpec((1, F), lambda b: (0, 0)),
            pl.BlockSpec((F, LBL), lambda b: (0, 0)),
            pl.BlockSpec((1, LBL), lambda b: (0, 0)),
        ],
        out_specs=pl.BlockSpec((1, N, LBL), lambda b: (b, 0, 0)),
        out_shape=jax.ShapeDtypeStruct((B, N, LBL), f32),
        scratch_shapes=[
            pltpu.VMEM((C, N, N), f32),
            pltpu.VMEM((C, N), f32),
            pltpu.VMEM((N, C), f32),
            pltpu.VMEM((N, N), f32),
            pltpu.VMEM((N, N), f32),
            pltpu.VMEM((N, C), f32),
            pltpu.VMEM((C, N), f32),
        ],
    )(x, a, e2, et2, wsi, wsj, bs[None], wsc, wnx, wni, wnj,
      bn[None], Wd, bd[None])
    return out


# speedup vs baseline: 1.0261x; 1.0261x over previous
import jax
import jax.numpy as jnp
from jax import lax
from jax.experimental import pallas as pl
from jax.experimental.pallas import tpu as pltpu


def _net_body(x_ref, a_ref, e_ref, et_ref, wsi_ref, wsj_ref, bs_ref,
              wsc_ref, wnx_ref, wni_ref, wnj_ref, bn_ref, wd_ref, bd_ref,
              out_ref, s_scr, vt_scr, u_scr, di_scr, dj_scr, pi_scr, pjt_scr):
    N = a_ref.shape[1]
    C = bs_ref.shape[1]
    f32 = jnp.float32

    xb = x_ref[0]

    u_scr[...] = jnp.dot(xb, wsi_ref[...],
                         preferred_element_type=f32) + bs_ref[...]
    v = jnp.dot(xb, wsj_ref[...], preferred_element_type=f32)
    vt_scr[...] = v.T

    ones_col = jnp.ones((N, 1), f32)
    ones_row = jnp.ones((1, N), f32)
    iota_cc = lax.broadcasted_iota(jnp.int32, (C, 16), 0)
    iota_8c = lax.broadcasted_iota(jnp.int32, (16, C), 1)
    eight_col = lax.broadcasted_iota(jnp.int32, (C, 16), 1)
    eight_row = lax.broadcasted_iota(jnp.int32, (16, C), 0)

    TI = 40

    def pass1_tile(t, _):
        i0 = t * TI
        for j0 in range(0, N, 128):
            W = min(128, N - j0)
            eb = e_ref[0, pl.ds(i0, TI), j0:j0 + W]
            etb = et_ref[0, pl.ds(i0, TI), j0:j0 + W]
            acc_i = None
            for c in range(C):
                ub = jnp.broadcast_to(u_scr[pl.ds(i0, TI), c:c + 1],
                                      (TI, W))
                vb = jnp.broadcast_to(vt_scr[c:c + 1, j0:j0 + W], (TI, W))
                s_c = jnp.maximum(ub + vb + wsc_ref[0, c] * eb
                                  + wsc_ref[1, c] * etb, 0.0)
                s_scr[c, pl.ds(i0, TI), j0:j0 + W] = s_c
                if acc_i is None:
                    acc_i = wsc_ref[2, c] * s_c
                    acc_j = wsc_ref[3, c] * s_c
                else:
                    acc_i = acc_i + wsc_ref[2, c] * s_c
                    acc_j = acc_j + wsc_ref[3, c] * s_c
            di_scr[pl.ds(i0, TI), j0:j0 + W] = acc_i
            dj_scr[pl.ds(i0, TI), j0:j0 + W] = acc_j
        return 0

    lax.fori_loop(0, N // TI, pass1_tile, 0)

    bai = wsc_ref[4, 0]
    baj = wsc_ref[4, 1]
    ab_full = a_ref[0]
    di_scr[...] = ab_full * jax.nn.sigmoid(ab_full * di_scr[...] + bai)
    dj_scr[...] = ab_full * jax.nn.sigmoid(ab_full * dj_scr[...] + baj)

    pi_scr[...] = jnp.zeros((N, C), f32)
    pjt_scr[...] = jnp.zeros((C, N), f32)

    def pass2(c0):
        sigi = di_scr[...]
        sigj = dj_scr[...]
        pcl = []
        prl = []
        for k in range(16):
            s_k = s_scr[c0 + k]
            pcl.append(jnp.dot(s_k * sigi, ones_col,
                               preferred_element_type=f32))
            prl.append(jnp.dot(ones_row, s_k * sigj,
                               preferred_element_type=f32))
        pcols = jnp.concatenate(pcl, axis=1)
        prows = jnp.concatenate(prl, axis=0)
        ohrows = (iota_8c == c0 + eight_row).astype(f32)
        pi_scr[...] = pi_scr[...] + jnp.dot(pcols, ohrows,
                                            preferred_element_type=f32)
        pjt_scr[...] = pjt_scr[...] + lax.dot_general(
            ohrows, prows, (((0,), (0,)), ((), ())),
            preferred_element_type=f32)

    pass2(0)
    pass2(16)

    xo = (jnp.dot(xb, wnx_ref[...], preferred_element_type=f32)
          + jnp.dot(pi_scr[...], wni_ref[...], preferred_element_type=f32)
          + lax.dot_general(pjt_scr[...], wnj_ref[...],
                            (((0,), (0,)), ((), ())),
                            preferred_element_type=f32)
          + bn_ref[...])
    out_ref[0] = jnp.dot(xo, wd_ref[...], preferred_element_type=f32) \
        + bd_ref[...]


def kernel(x, a, e, Ws, bs, Wai, bai, Waj, baj, Wn, bn, We, be, Wd, bd):
    B, N, F = x.shape
    C = Ws.shape[1]
    LBL = Wd.shape[1]
    f32 = jnp.float32

    e2 = e[..., 0]
    et2 = jnp.swapaxes(e2, 1, 2)
    wsi = Ws[:F]
    wsj = Ws[F:2 * F]
    brow = jnp.zeros((C,), f32).at[0].set(bai[0]).at[1].set(baj[0])
    wsc = jnp.stack([Ws[2 * F], Ws[2 * F + 1], Wai[:, 0], Waj[:, 0], brow],
                    axis=0)
    wnx = Wn[:F]
    wni = Wn[F:F + C]
    wnj = Wn[F + C:]

    out = pl.pallas_call(
        _net_body,
        grid=(B,),
        in_specs=[
            pl.BlockSpec((1, N, F), lambda b: (b, 0, 0)),
            pl.BlockSpec((1, N, N), lambda b: (b, 0, 0)),
            pl.BlockSpec((1, N, N), lambda b: (b, 0, 0)),
            pl.BlockSpec((1, N, N), lambda b: (b, 0, 0)),
            pl.BlockSpec((F, C), lambda b: (0, 0)),
            pl.BlockSpec((F, C), lambda b: (0, 0)),
            pl.BlockSpec((1, C), lambda b: (0, 0)),
            pl.BlockSpec(memory_space=pltpu.SMEM),
            pl.BlockSpec((F, F), lambda b: (0, 0)),
            pl.BlockSpec((C, F), lambda b: (0, 0)),
            pl.BlockSpec((C, F), lambda b: (0, 0)),
            pl.BlockSpec((1, F), lambda b: (0, 0)),
            pl.BlockSpec((F, LBL), lambda b: (0, 0)),
            pl.BlockSpec((1, LBL), lambda b: (0, 0)),
        ],
        out_specs=pl.BlockSpec((1, N, LBL), lambda b: (b, 0, 0)),
        out_shape=jax.ShapeDtypeStruct((B, N, LBL), f32),
        scratch_shapes=[
            pltpu.VMEM((C, N, N), f32),
            pltpu.VMEM((C, N), f32),
            pltpu.VMEM((N, C), f32),
            pltpu.VMEM((N, N), f32),
            pltpu.VMEM((N, N), f32),
            pltpu.VMEM((N, C), f32),
            pltpu.VMEM((C, N), f32),
        ],
    )(x, a, e2, et2, wsi, wsj, bs[None], wsc, wnx, wni, wnj,
      bn[None], Wd, bd[None])
    return out


# speedup vs baseline: 1.0515x; 1.0248x over previous
import jax
import jax.numpy as jnp
from jax import lax
from jax.experimental import pallas as pl
from jax.experimental.pallas import tpu as pltpu


def _net_body(x_ref, a_ref, e_ref, et_ref, wsi_ref, wsj_ref, bs_ref,
              wsc_ref, wnx_ref, wni_ref, wnj_ref, bn_ref, wd_ref, bd_ref,
              out_ref, s_scr, vt_scr, u_scr, di_scr, dj_scr, pi_scr, pjt_scr):
    N = a_ref.shape[1]
    C = bs_ref.shape[1]
    f32 = jnp.float32

    xb = x_ref[0]

    u_scr[...] = jnp.dot(xb, wsi_ref[...],
                         preferred_element_type=f32) + bs_ref[...]
    v = jnp.dot(xb, wsj_ref[...], preferred_element_type=f32)
    vt_scr[...] = v.T

    ones_col = jnp.ones((N, 1), f32)
    ones_row = jnp.ones((1, N), f32)
    iota_cc = lax.broadcasted_iota(jnp.int32, (C, 16), 0)
    iota_8c = lax.broadcasted_iota(jnp.int32, (16, C), 1)
    eight_col = lax.broadcasted_iota(jnp.int32, (C, 16), 1)
    eight_row = lax.broadcasted_iota(jnp.int32, (16, C), 0)

    TI = 80

    def pass1_tile(t, _):
        i0 = t * TI
        for j0 in range(0, N, 128):
            W = min(128, N - j0)
            eb = e_ref[0, pl.ds(i0, TI), j0:j0 + W]
            etb = et_ref[0, pl.ds(i0, TI), j0:j0 + W]
            acc_i = None
            for c in range(C):
                ub = jnp.broadcast_to(u_scr[pl.ds(i0, TI), c:c + 1],
                                      (TI, W))
                vb = jnp.broadcast_to(vt_scr[c:c + 1, j0:j0 + W], (TI, W))
                s_c = jnp.maximum(ub + vb + wsc_ref[0, c] * eb
                                  + wsc_ref[1, c] * etb, 0.0)
                s_scr[c, pl.ds(i0, TI), j0:j0 + W] = s_c
                if acc_i is None:
                    acc_i = wsc_ref[2, c] * s_c
                    acc_j = wsc_ref[3, c] * s_c
                else:
                    acc_i = acc_i + wsc_ref[2, c] * s_c
                    acc_j = acc_j + wsc_ref[3, c] * s_c
            di_scr[pl.ds(i0, TI), j0:j0 + W] = acc_i
            dj_scr[pl.ds(i0, TI), j0:j0 + W] = acc_j
        return 0

    lax.fori_loop(0, N // TI, pass1_tile, 0)

    bai = wsc_ref[4, 0]
    baj = wsc_ref[4, 1]
    ab_full = a_ref[0]
    di_scr[...] = ab_full * jax.nn.sigmoid(ab_full * di_scr[...] + bai)
    dj_scr[...] = ab_full * jax.nn.sigmoid(ab_full * dj_scr[...] + baj)

    pi_scr[...] = jnp.zeros((N, C), f32)
    pjt_scr[...] = jnp.zeros((C, N), f32)

    def pass2(c0):
        sigi = di_scr[...]
        sigj = dj_scr[...]
        pcl = []
        prl = []
        for k in range(16):
            s_k = s_scr[c0 + k]
            pcl.append(jnp.dot(s_k * sigi, ones_col,
                               preferred_element_type=f32))
            prl.append(jnp.dot(ones_row, s_k * sigj,
                               preferred_element_type=f32))
        pcols = jnp.concatenate(pcl, axis=1)
        prows = jnp.concatenate(prl, axis=0)
        ohrows = (iota_8c == c0 + eight_row).astype(f32)
        pi_scr[...] = pi_scr[...] + jnp.dot(pcols, ohrows,
                                            preferred_element_type=f32)
        pjt_scr[...] = pjt_scr[...] + lax.dot_general(
            ohrows, prows, (((0,), (0,)), ((), ())),
            preferred_element_type=f32)

    pass2(0)
    pass2(16)

    xo = (jnp.dot(xb, wnx_ref[...], preferred_element_type=f32)
          + jnp.dot(pi_scr[...], wni_ref[...], preferred_element_type=f32)
          + lax.dot_general(pjt_scr[...], wnj_ref[...],
                            (((0,), (0,)), ((), ())),
                            preferred_element_type=f32)
          + bn_ref[...])
    out_ref[0] = jnp.dot(xo, wd_ref[...], preferred_element_type=f32) \
        + bd_ref[...]


def kernel(x, a, e, Ws, bs, Wai, bai, Waj, baj, Wn, bn, We, be, Wd, bd):
    B, N, F = x.shape
    C = Ws.shape[1]
    LBL = Wd.shape[1]
    f32 = jnp.float32

    e2 = e[..., 0]
    et2 = jnp.swapaxes(e2, 1, 2)
    wsi = Ws[:F]
    wsj = Ws[F:2 * F]
    brow = jnp.zeros((C,), f32).at[0].set(bai[0]).at[1].set(baj[0])
    wsc = jnp.stack([Ws[2 * F], Ws[2 * F + 1], Wai[:, 0], Waj[:, 0], brow],
                    axis=0)
    wnx = Wn[:F]
    wni = Wn[F:F + C]
    wnj = Wn[F + C:]

    out = pl.pallas_call(
        _net_body,
        grid=(B,),
        in_specs=[
            pl.BlockSpec((1, N, F), lambda b: (b, 0, 0)),
            pl.BlockSpec((1, N, N), lambda b: (b, 0, 0)),
            pl.BlockSpec((1, N, N), lambda b: (b, 0, 0)),
            pl.BlockSpec((1, N, N), lambda b: (b, 0, 0)),
            pl.BlockSpec((F, C), lambda b: (0, 0)),
            pl.BlockSpec((F, C), lambda b: (0, 0)),
            pl.BlockSpec((1, C), lambda b: (0, 0)),
            pl.BlockSpec(memory_space=pltpu.SMEM),
            pl.BlockSpec((F, F), lambda b: (0, 0)),
            pl.BlockSpec((C, F), lambda b: (0, 0)),
            pl.BlockSpec((C, F), lambda b: (0, 0)),
            pl.BlockSpec((1, F), lambda b: (0, 0)),
            pl.BlockSpec((F, LBL), lambda b: (0, 0)),
            pl.BlockSpec((1, LBL), lambda b: (0, 0)),
        ],
        out_specs=pl.BlockSpec((1, N, LBL), lambda b: (b, 0, 0)),
        out_shape=jax.ShapeDtypeStruct((B, N, LBL), f32),
        scratch_shapes=[
            pltpu.VMEM((C, N, N), f32),
            pltpu.VMEM((C, N), f32),
            pltpu.VMEM((N, C), f32),
            pltpu.VMEM((N, N), f32),
            pltpu.VMEM((N, N), f32),
            pltpu.VMEM((N, C), f32),
            pltpu.VMEM((C, N), f32),
        ],
    )(x, a, e2, et2, wsi, wsj, bs[None], wsc, wnx, wni, wnj,
      bn[None], Wd, bd[None])
    return out


# speedup vs baseline: 1.0638x; 1.0117x over previous
import jax
import jax.numpy as jnp
from jax import lax
from jax.experimental import pallas as pl
from jax.experimental.pallas import tpu as pltpu


def _net_body(x_ref, a_ref, e_ref, et_ref, wsi_ref, wsj_ref, bs_ref,
              wsc_ref, wnx_ref, wni_ref, wnj_ref, bn_ref, wd_ref, bd_ref,
              out_ref, s_scr, vt_scr, u_scr, di_scr, dj_scr, pi_scr, pjt_scr):
    N = a_ref.shape[1]
    C = bs_ref.shape[1]
    f32 = jnp.float32

    xb = x_ref[0]

    u_scr[...] = jnp.dot(xb, wsi_ref[...],
                         preferred_element_type=f32) + bs_ref[...]
    v = jnp.dot(xb, wsj_ref[...], preferred_element_type=f32)
    vt_scr[...] = v.T

    ones_col = jnp.ones((N, 1), f32)
    ones_row = jnp.ones((1, N), f32)
    iota_cc = lax.broadcasted_iota(jnp.int32, (C, 16), 0)
    iota_8c = lax.broadcasted_iota(jnp.int32, (16, C), 1)
    eight_col = lax.broadcasted_iota(jnp.int32, (C, 16), 1)
    eight_row = lax.broadcasted_iota(jnp.int32, (16, C), 0)

    TI = 80

    def pass1_tile(t, _):
        i0 = t * TI
        for j0 in range(0, N, 128):
            W = min(128, N - j0)
            eb = e_ref[0, pl.ds(i0, TI), j0:j0 + W]
            etb = et_ref[0, pl.ds(i0, TI), j0:j0 + W]
            acc_i = None
            for c in range(C):
                ub = jnp.broadcast_to(u_scr[pl.ds(i0, TI), c:c + 1],
                                      (TI, W))
                vb = jnp.broadcast_to(vt_scr[c:c + 1, j0:j0 + W], (TI, W))
                s_c = jnp.maximum(ub + vb + wsc_ref[0, c] * eb
                                  + wsc_ref[1, c] * etb, 0.0)
                s_scr[c, pl.ds(i0, TI), j0:j0 + W] = s_c
                if acc_i is None:
                    acc_i = wsc_ref[2, c] * s_c
                    acc_j = wsc_ref[3, c] * s_c
                else:
                    acc_i = acc_i + wsc_ref[2, c] * s_c
                    acc_j = acc_j + wsc_ref[3, c] * s_c
            di_scr[pl.ds(i0, TI), j0:j0 + W] = acc_i
            dj_scr[pl.ds(i0, TI), j0:j0 + W] = acc_j
        return 0

    lax.fori_loop(0, N // TI, pass1_tile, 0)

    bai = wsc_ref[4, 0]
    baj = wsc_ref[4, 1]
    ab_full = a_ref[0]
    di_scr[...] = ab_full * jax.nn.sigmoid(ab_full * di_scr[...] + bai)
    dj_scr[...] = ab_full * jax.nn.sigmoid(ab_full * dj_scr[...] + baj)

    sigi = di_scr[...]
    sigj = dj_scr[...]
    pcl = []
    prl = []
    for c in range(C):
        s_c = s_scr[c]
        pcl.append(jnp.dot(s_c * sigi, ones_col,
                           preferred_element_type=f32))
        prl.append(jnp.dot(ones_row, s_c * sigj,
                           preferred_element_type=f32))
    pi_scr[...] = jnp.concatenate(pcl, axis=1)
    pjt_scr[...] = jnp.concatenate(prl, axis=0)

    xo = (jnp.dot(xb, wnx_ref[...], preferred_element_type=f32)
          + jnp.dot(pi_scr[...], wni_ref[...], preferred_element_type=f32)
          + lax.dot_general(pjt_scr[...], wnj_ref[...],
                            (((0,), (0,)), ((), ())),
                            preferred_element_type=f32)
          + bn_ref[...])
    out_ref[0] = jnp.dot(xo, wd_ref[...], preferred_element_type=f32) \
        + bd_ref[...]


def kernel(x, a, e, Ws, bs, Wai, bai, Waj, baj, Wn, bn, We, be, Wd, bd):
    B, N, F = x.shape
    C = Ws.shape[1]
    LBL = Wd.shape[1]
    f32 = jnp.float32

    e2 = e[..., 0]
    et2 = jnp.swapaxes(e2, 1, 2)
    wsi = Ws[:F]
    wsj = Ws[F:2 * F]
    brow = jnp.zeros((C,), f32).at[0].set(bai[0]).at[1].set(baj[0])
    wsc = jnp.stack([Ws[2 * F], Ws[2 * F + 1], Wai[:, 0], Waj[:, 0], brow],
                    axis=0)
    wnx = Wn[:F]
    wni = Wn[F:F + C]
    wnj = Wn[F + C:]

    out = pl.pallas_call(
        _net_body,
        grid=(B,),
        in_specs=[
            pl.BlockSpec((1, N, F), lambda b: (b, 0, 0)),
            pl.BlockSpec((1, N, N), lambda b: (b, 0, 0)),
            pl.BlockSpec((1, N, N), lambda b: (b, 0, 0)),
            pl.BlockSpec((1, N, N), lambda b: (b, 0, 0)),
            pl.BlockSpec((F, C), lambda b: (0, 0)),
            pl.BlockSpec((F, C), lambda b: (0, 0)),
            pl.BlockSpec((1, C), lambda b: (0, 0)),
            pl.BlockSpec(memory_space=pltpu.SMEM),
            pl.BlockSpec((F, F), lambda b: (0, 0)),
            pl.BlockSpec((C, F), lambda b: (0, 0)),
            pl.BlockSpec((C, F), lambda b: (0, 0)),
            pl.BlockSpec((1, F), lambda b: (0, 0)),
            pl.BlockSpec((F, LBL), lambda b: (0, 0)),
            pl.BlockSpec((1, LBL), lambda b: (0, 0)),
        ],
        out_specs=pl.BlockSpec((1, N, LBL), lambda b: (b, 0, 0)),
        out_shape=jax.ShapeDtypeStruct((B, N, LBL), f32),
        scratch_shapes=[
            pltpu.VMEM((C, N, N), f32),
            pltpu.VMEM((C, N), f32),
            pltpu.VMEM((N, C), f32),
            pltpu.VMEM((N, N), f32),
            pltpu.VMEM((N, N), f32),
            pltpu.VMEM((N, C), f32),
            pltpu.VMEM((C, N), f32),
        ],
    )(x, a, e2, et2, wsi, wsj, bs[None], wsc, wnx, wni, wnj,
      bn[None], Wd, bd[None])
    return out
